# Initial kernel scaffold; baseline (speedup 1.0000x reference)
import jax
import jax.numpy as jnp
from jax.experimental import pallas as pl


def kernel(x, indices, output_size):
    raise NotImplementedError("write your pallas kernel here")


# speedup vs baseline: 63.1815x; 63.1815x over previous
import functools

import jax
import jax.numpy as jnp
from jax import lax
from jax.experimental import pallas as pl
from jax.experimental.pallas import tpu as pltpu
from jax.experimental.pallas import tpu_sc as plsc

_B, _C, _H, _W = 8, 256, 64, 64
_Ho, _Wo = 128, 128
_P = _B * _C
_IN = _H * _W
_OUT = _Ho * _Wo
_NW = 32
_PPW = _P // _NW
_CH = _IN // 16


def _unpool_sc(x2, i2):
    mesh = plsc.VectorSubcoreMesh(core_axis_name="c", subcore_axis_name="s")

    @functools.partial(
        pl.kernel,
        mesh=mesh,
        compiler_params=pltpu.CompilerParams(needs_layout_passes=False),
        out_type=jax.ShapeDtypeStruct((_P, _OUT), jnp.float32),
        scratch_types=[
            pltpu.VMEM((_IN,), jnp.float32),
            pltpu.VMEM((_IN,), jnp.int32),
            pltpu.VMEM((_OUT,), jnp.float32),
        ],
    )
    def body(x_hbm, i_hbm, o_hbm, x_v, i_v, o_v):
        w = lax.axis_index("s") * 2 + lax.axis_index("c")
        zf = jnp.zeros((16,), jnp.float32)

        def zero_body(k, carry):
            o_v[pl.ds(k * 16, 16)] = zf
            return carry

        lax.fori_loop(0, _OUT // 16, zero_body, 0)

        def plane_body(t, carry):
            p = w * _PPW + t
            pltpu.sync_copy(x_hbm.at[p], x_v)
            pltpu.sync_copy(i_hbm.at[p], i_v)

            def scat(k, c):
                iv = i_v[pl.ds(k * 16, 16)]
                xv = x_v[pl.ds(k * 16, 16)]
                plsc.store_scatter(o_v, [iv], xv)
                return c

            lax.fori_loop(0, _CH, scat, 0)
            pltpu.sync_copy(o_v, o_hbm.at[p])

            def unscat(k, c):
                iv = i_v[pl.ds(k * 16, 16)]
                plsc.store_scatter(o_v, [iv], zf)
                return c

            lax.fori_loop(0, _CH, unscat, 0)
            return carry

        lax.fori_loop(0, _PPW, plane_body, 0)

    return body(x2, i2)


def kernel(x, indices, output_size):
    del output_size
    x2 = x.reshape(_P, _IN)
    i2 = indices.reshape(_P, _IN)
    out = _unpool_sc(x2, i2)
    return out.reshape(_B, _C, _Ho, _Wo)


# speedup vs baseline: 90.3203x; 1.4295x over previous
import functools

import jax
import jax.numpy as jnp
from jax import lax
from jax.experimental import pallas as pl
from jax.experimental.pallas import tpu as pltpu
from jax.experimental.pallas import tpu_sc as plsc

_B, _C, _H, _W = 8, 256, 64, 64
_Ho, _Wo = 128, 128
_P = _B * _C
_IN = _H * _W
_OUT = _Ho * _Wo
_NW = 32
_PPW = _P // _NW
_GRP = 4
_UNROLL = 8
_CH = _IN // 16


def _unpool_sc(x2, i2):
    mesh = plsc.VectorSubcoreMesh(core_axis_name="c", subcore_axis_name="s")

    @functools.partial(
        pl.kernel,
        mesh=mesh,
        compiler_params=pltpu.CompilerParams(needs_layout_passes=False),
        out_type=jax.ShapeDtypeStruct((_B, _C, _Ho, _Wo), jnp.float32),
        scratch_types=[
            pltpu.VMEM((_GRP, _IN), jnp.float32),
            pltpu.VMEM((_GRP, _IN), jnp.int32),
            pltpu.VMEM((_Ho, _Wo), jnp.float32),
        ],
    )
    def body(x_hbm, i_hbm, o_hbm, x_v, i_v, o_v):
        w = lax.axis_index("s") * 2 + lax.axis_index("c")
        zf = jnp.zeros((16,), jnp.float32)

        def zero_body(k, carry):
            o_v[k >> 3, pl.ds((k & 7) * 16, 16)] = zf
            return carry

        lax.fori_loop(0, _OUT // 16, zero_body, 0)

        def group_body(g, carry):
            p0 = w * _PPW + g * _GRP
            pltpu.sync_copy(x_hbm.at[pl.ds(p0, _GRP)], x_v)
            pltpu.sync_copy(i_hbm.at[pl.ds(p0, _GRP)], i_v)
            for s in range(_GRP):
                p = p0 + s
                bi = p >> 8
                ci = p & 255

                def scat(kk, c, s=s):
                    for u in range(_UNROLL):
                        off = kk * (16 * _UNROLL) + u * 16
                        iv = i_v[s, pl.ds(off, 16)]
                        xv = x_v[s, pl.ds(off, 16)]
                        plsc.store_scatter(o_v, [iv >> 7, iv & 127], xv)
                    return c

                lax.fori_loop(0, _CH // _UNROLL, scat, 0)
                pltpu.sync_copy(o_v, o_hbm.at[bi, ci])

                def unscat(kk, c, s=s):
                    for u in range(_UNROLL):
                        off = kk * (16 * _UNROLL) + u * 16
                        iv = i_v[s, pl.ds(off, 16)]
                        plsc.store_scatter(o_v, [iv >> 7, iv & 127], zf)
                    return c

                lax.fori_loop(0, _CH // _UNROLL, unscat, 0)
            return carry

        lax.fori_loop(0, _PPW // _GRP, group_body, 0)

    return body(x2, i2)


def kernel(x, indices, output_size):
    del output_size
    x2 = x.reshape(_P, _IN)
    i2 = indices.reshape(_P, _IN)
    return _unpool_sc(x2, i2)


# speedup vs baseline: 92.8227x; 1.0277x over previous
import functools

import jax
import jax.numpy as jnp
from jax import lax
from jax.experimental import pallas as pl
from jax.experimental.pallas import tpu as pltpu
from jax.experimental.pallas import tpu_sc as plsc

_B, _C, _H, _W = 8, 256, 64, 64
_Ho, _Wo = 128, 128
_P = _B * _C
_IN = _H * _W
_OUT = _Ho * _Wo
_NW = 32
_PPW = _P // _NW
_GRP = 4
_UNROLL = 8
_CH = _IN // 16


def _unpool_sc(x4, i4):
    mesh = plsc.VectorSubcoreMesh(core_axis_name="c", subcore_axis_name="s")

    @functools.partial(
        pl.kernel,
        mesh=mesh,
        compiler_params=pltpu.CompilerParams(needs_layout_passes=False),
        out_type=jax.ShapeDtypeStruct((_B, _C, _Ho, _Wo), jnp.float32),
        scratch_types=[
            pltpu.VMEM((_GRP, _H, _W), jnp.float32),
            pltpu.VMEM((_GRP, _H, _W), jnp.int32),
            pltpu.VMEM((_Ho, _Wo), jnp.float32),
        ],
    )
    def body(x_hbm, i_hbm, o_hbm, x_v, i_v, o_v):
        w = lax.axis_index("s") * 2 + lax.axis_index("c")
        zf = jnp.zeros((16,), jnp.float32)

        def zero_body(k, carry):
            o_v[k >> 3, pl.ds((k & 7) * 16, 16)] = zf
            return carry

        lax.fori_loop(0, _OUT // 16, zero_body, 0)

        def group_body(g, carry):
            p0 = w * _PPW + g * _GRP
            bi = p0 >> 8
            c0 = p0 & 255
            pltpu.sync_copy(x_hbm.at[bi, pl.ds(c0, _GRP)], x_v)
            pltpu.sync_copy(i_hbm.at[bi, pl.ds(c0, _GRP)], i_v)
            for s in range(_GRP):
                def scat(kk, c, s=s):
                    for u in range(_UNROLL):
                        r = kk * (_UNROLL // 4) + (u >> 2)
                        col = (u & 3) * 16
                        iv = i_v[s, r, pl.ds(col, 16)]
                        xv = x_v[s, r, pl.ds(col, 16)]
                        plsc.store_scatter(o_v, [iv >> 7, iv & 127], xv)
                    return c

                lax.fori_loop(0, _CH // _UNROLL, scat, 0)
                pltpu.sync_copy(o_v, o_hbm.at[bi, c0 + s])

                def unscat(kk, c, s=s):
                    for u in range(_UNROLL):
                        r = kk * (_UNROLL // 4) + (u >> 2)
                        col = (u & 3) * 16
                        iv = i_v[s, r, pl.ds(col, 16)]
                        plsc.store_scatter(o_v, [iv >> 7, iv & 127], zf)
                    return c

                lax.fori_loop(0, _CH // _UNROLL, unscat, 0)
            return carry

        lax.fori_loop(0, _PPW // _GRP, group_body, 0)

    return body(x4, i4)


def kernel(x, indices, output_size):
    del output_size
    return _unpool_sc(x, indices)


# speedup vs baseline: 142.6657x; 1.5370x over previous
import functools

import jax
import jax.numpy as jnp
from jax import lax
from jax.experimental import pallas as pl
from jax.experimental.pallas import tpu as pltpu
from jax.experimental.pallas import tpu_sc as plsc

_B, _C, _H, _W = 8, 256, 64, 64
_Ho, _Wo = 128, 128
_P = _B * _C
_NW = 32
_PPW = _P // _NW
_GRP = 2
_NGRP = _PPW // _GRP
_UNROLL = 8
_CH = (_H * _W) // 16


def _unpool_sc(x4, i4):
    mesh = plsc.VectorSubcoreMesh(core_axis_name="c", subcore_axis_name="s")

    @functools.partial(
        pl.kernel,
        mesh=mesh,
        compiler_params=pltpu.CompilerParams(needs_layout_passes=False),
        out_type=jax.ShapeDtypeStruct((_B, _C, _Ho, _Wo), jnp.float32),
        scratch_types=[
            pltpu.VMEM((_GRP, _H, _W), jnp.float32),
            pltpu.VMEM((_GRP, _H, _W), jnp.float32),
            pltpu.VMEM((_GRP, _H, _W), jnp.int32),
            pltpu.VMEM((_GRP, _H, _W), jnp.int32),
            pltpu.VMEM((_Ho, _Wo), jnp.float32),
            pltpu.VMEM((_Ho, _Wo), jnp.float32),
            pltpu.SemaphoreType.DMA,
            pltpu.SemaphoreType.DMA,
            pltpu.SemaphoreType.DMA,
            pltpu.SemaphoreType.DMA,
        ],
    )
    def body(x_hbm, i_hbm, o_hbm, x_v0, x_v1, i_v0, i_v1, o_v0, o_v1,
             sem_in0, sem_in1, sem_f0, sem_f1):
        w = lax.axis_index("s") * 2 + lax.axis_index("c")
        bi = w >> 2
        cbase = (w & 3) * _PPW
        zf = jnp.zeros((16,), jnp.float32)
        xbufs, ibufs = [x_v0, x_v1], [i_v0, i_v1]
        obufs = [o_v0, o_v1]
        isems, fsems = [sem_in0, sem_in1], [sem_f0, sem_f1]

        def issue_group(gb, g):
            c0 = cbase + g * _GRP
            pltpu.async_copy(x_hbm.at[bi, pl.ds(c0, _GRP)], xbufs[gb], isems[gb])
            pltpu.async_copy(i_hbm.at[bi, pl.ds(c0, _GRP)], ibufs[gb], isems[gb])

        def wait_group(gb):
            pltpu.make_async_copy(x_hbm.at[bi, pl.ds(cbase, _GRP)],
                                  xbufs[gb], isems[gb]).wait()
            pltpu.make_async_copy(i_hbm.at[bi, pl.ds(cbase, _GRP)],
                                  ibufs[gb], isems[gb]).wait()

        def flush(sb, ci):
            pltpu.async_copy(obufs[sb], o_hbm.at[bi, ci], fsems[sb])

        def flush_wait(sb, ci):
            pltpu.make_async_copy(obufs[sb], o_hbm.at[bi, ci],
                                  fsems[sb]).wait()

        def scat(gb, sg, ob):
            xb, ib = xbufs[gb], ibufs[gb]

            def kbody(kk, c):
                ivs, xvs = [], []
                for u in range(_UNROLL):
                    r = kk * (_UNROLL // 4) + (u >> 2)
                    col = (u & 3) * 16
                    ivs.append(ib[sg, r, pl.ds(col, 16)])
                    xvs.append(xb[sg, r, pl.ds(col, 16)])
                for u in range(_UNROLL):
                    plsc.store_scatter(ob, [ivs[u] >> 7, ivs[u] & 127], xvs[u])
                return c

            lax.fori_loop(0, _CH // _UNROLL, kbody, 0)

        def unscat(gb, sg, ob):
            ib = ibufs[gb]

            def kbody(kk, c):
                ivs = []
                for u in range(_UNROLL):
                    r = kk * (_UNROLL // 4) + (u >> 2)
                    col = (u & 3) * 16
                    ivs.append(ib[sg, r, pl.ds(col, 16)])
                for u in range(_UNROLL):
                    plsc.store_scatter(ob, [ivs[u] >> 7, ivs[u] & 127], zf)
                return c

            lax.fori_loop(0, _CH // _UNROLL, kbody, 0)

        issue_group(0, 0)

        def zero_body(k, carry):
            o_v0[k >> 3, pl.ds((k & 7) * 16, 16)] = zf
            o_v1[k >> 3, pl.ds((k & 7) * 16, 16)] = zf
            return carry

        lax.fori_loop(0, (_Ho * _Wo) // 16, zero_body, 0)

        for s in range(4):
            gb, sb = s // 2, s % 2
            ci = cbase + s
            if s == 0:
                wait_group(0)
                issue_group(1, 1)
            if s == 2:
                wait_group(1)
            if s >= 2:
                flush_wait(sb, ci - 2)
                unscat(0, s - 2, obufs[sb])
            scat(gb, s % 2, obufs[sb])
            flush(sb, ci)
            if s == 3:
                issue_group(0, 2)

        def steady(m, carry):
            qmap = [(1, 0), (1, 1), (0, 0), (0, 1)]
            for s in range(4):
                gb, sb = s // 2, s % 2
                ci = cbase + 4 * m + s
                if s == 0:
                    wait_group(0)
                if s == 2:
                    wait_group(1)
                q_gb, q_sg = qmap[s]
                flush_wait(sb, ci - 2)
                unscat(q_gb, q_sg, obufs[sb])
                scat(gb, s % 2, obufs[sb])
                flush(sb, ci)
                if s == 1:
                    issue_group(1, 2 * m + 1)
                if s == 3:
                    issue_group(0, lax.min(2 * m + 2, _NGRP - 1))
            return carry

        lax.fori_loop(1, _PPW // 4, steady, 0)

        flush_wait(0, cbase + _PPW - 2)
        flush_wait(1, cbase + _PPW - 1)
        wait_group(0)

    return body(x4, i4)


def kernel(x, indices, output_size):
    del output_size
    return _unpool_sc(x, indices)


# speedup vs baseline: 171.5647x; 1.2026x over previous
import functools

import jax
import jax.numpy as jnp
from jax import lax
from jax.experimental import pallas as pl
from jax.experimental.pallas import tpu as pltpu
from jax.experimental.pallas import tpu_sc as plsc

_B, _C, _H, _W = 8, 256, 64, 64
_Ho, _Wo = 128, 128
_NW = 32
_RPW = (_B * _H) // _NW
_NBLK = _RPW * 2
_CB = _C // 2
_NCHB = _CB // 16


def _unpool_sc(xt, it):
    mesh = plsc.VectorSubcoreMesh(core_axis_name="c", subcore_axis_name="s")

    @functools.partial(
        pl.kernel,
        mesh=mesh,
        compiler_params=pltpu.CompilerParams(needs_layout_passes=False),
        out_type=jax.ShapeDtypeStruct((_B, _C, _Ho, _Wo), jnp.float32),
        scratch_types=[
            pltpu.VMEM((_W, _CB), jnp.float32),
            pltpu.VMEM((_W, _CB), jnp.float32),
            pltpu.VMEM((_W, _CB), jnp.int32),
            pltpu.VMEM((_W, _CB), jnp.int32),
            pltpu.VMEM((_W, _CB), jnp.int32),
            pltpu.VMEM((_W, _CB), jnp.int32),
            pltpu.VMEM((_CB, 2, _Wo), jnp.float32),
            pltpu.VMEM((_CB, 2, _Wo), jnp.float32),
            pltpu.SemaphoreType.DMA,
            pltpu.SemaphoreType.DMA,
            pltpu.SemaphoreType.DMA,
            pltpu.SemaphoreType.DMA,
            pltpu.SemaphoreType.DMA,
            pltpu.SemaphoreType.DMA,
            pltpu.SemaphoreType.DMA,
            pltpu.SemaphoreType.DMA,
        ],
    )
    def body(x_hbm, i_hbm, o_hbm, x_v0, x_v1, i_v0, i_v1, i_v2, i_v3,
             st0, st1, smx0, smx1, smi0, smi1, smi2, smi3, smf0, smf1):
        w = lax.axis_index("s") * 2 + lax.axis_index("c")
        r0 = w * _RPW
        zf = jnp.zeros((16,), jnp.float32)
        iota = lax.iota(jnp.int32, 16)
        xbufs = [x_v0, x_v1]
        ibufs = [i_v0, i_v1, i_v2, i_v3]
        stbufs = [st0, st1]
        smx, smi, smf = [smx0, smx1], [smi0, smi1, smi2, smi3], [smf0, smf1]

        def blk_coords(q):
            rid = r0 + (q >> 1)
            return rid >> 6, rid & 63

        def issue_in(q, h, xb, ib):
            bq, iq = blk_coords(q)
            c0 = h * _CB
            pltpu.async_copy(x_hbm.at[bq, iq, pl.ds(0, _W), pl.ds(c0, _CB)],
                             xbufs[xb], smx[xb])
            pltpu.async_copy(i_hbm.at[bq, iq, pl.ds(0, _W), pl.ds(c0, _CB)],
                             ibufs[ib], smi[ib])

        def wait_in(xb, ib):
            pltpu.make_async_copy(
                x_hbm.at[0, 0, pl.ds(0, _W), pl.ds(0, _CB)],
                xbufs[xb], smx[xb]).wait()
            pltpu.make_async_copy(
                i_hbm.at[0, 0, pl.ds(0, _W), pl.ds(0, _CB)],
                ibufs[ib], smi[ib]).wait()

        def out_slice(q, h):
            bq, iq = blk_coords(q)
            return o_hbm.at[bq, pl.ds(h * _CB, _CB), pl.ds(2 * iq, 2),
                            pl.ds(0, _Wo)]

        def flush(sb, q, h):
            pltpu.async_copy(stbufs[sb], out_slice(q, h), smf[sb])

        def flush_wait(sb, q, h):
            pltpu.make_async_copy(stbufs[sb], out_slice(q, h), smf[sb]).wait()

        def scat_like(q, ib, sb, xb):
            _, iq = blk_coords(q)
            ir, st = ibufs[ib], stbufs[sb]
            xr = None if xb is None else xbufs[xb]
            ri2 = 2 * iq

            def kbody(j, c):
                ivs, xvs = [], []
                for u in range(_NCHB):
                    ivs.append(ir[j, pl.ds(u * 16, 16)])
                    if xr is not None:
                        xvs.append(xr[j, pl.ds(u * 16, 16)])
                for u in range(_NCHB):
                    cvec = iota + u * 16
                    drv = (ivs[u] >> 7) - ri2
                    dcv = ivs[u] & 127
                    val = zf if xr is None else xvs[u]
                    plsc.store_scatter(st, [cvec, drv, dcv], val)
                return c

            lax.fori_loop(0, _W, kbody, 0)

        issue_in(0, 0, 0, 0)

        def zero_body(k, carry):
            st0[k >> 4, (k >> 3) & 1, pl.ds((k & 7) * 16, 16)] = zf
            st1[k >> 4, (k >> 3) & 1, pl.ds((k & 7) * 16, 16)] = zf
            return carry

        lax.fori_loop(0, (_CB * 2 * _Wo) // 16, zero_body, 0)

        def run_block(q, s, peeled):
            h = s & 1
            xb, ib, sb = s % 2, s % 4, s % 2
            wait_in(xb, ib)
            qn = lax.min(q + 1, _NBLK - 1)
            issue_in(qn, (s + 1) & 1, (s + 1) % 2, (s + 1) % 4)
            if not (peeled and s < 2):
                flush_wait(sb, q - 2, h)
                scat_like(q - 2, (s + 2) % 4, sb, None)
            scat_like(q, ib, sb, xb)
            flush(sb, q, h)

        for s in range(4):
            run_block(s, s, peeled=True)

        def steady(mi, carry):
            for s in range(4):
                run_block(4 * mi + s, s, peeled=False)
            return carry

        lax.fori_loop(1, _NBLK // 4, steady, 0)

        flush_wait(0, _NBLK - 2, 0)
        flush_wait(1, _NBLK - 1, 1)
        pltpu.make_async_copy(x_hbm.at[0, 0, pl.ds(0, _W), pl.ds(0, _CB)],
                              xbufs[0], smx[0]).wait()
        pltpu.make_async_copy(i_hbm.at[0, 0, pl.ds(0, _W), pl.ds(0, _CB)],
                              ibufs[0], smi[0]).wait()

    return body(xt, it)


def kernel(x, indices, output_size):
    del output_size
    xt = x.transpose(0, 2, 3, 1)
    it = indices.transpose(0, 2, 3, 1)
    return _unpool_sc(xt, it)
